# Initial kernel scaffold; baseline (speedup 1.0000x reference)
import jax
import jax.numpy as jnp
from jax.experimental import pallas as pl


def kernel(box_prediction, class_prediction):
    raise NotImplementedError("write your pallas kernel here")


# speedup vs baseline: 7.5574x; 7.5574x over previous
import functools

import jax
import jax.numpy as jnp
from jax import lax
from jax.experimental import pallas as pl
from jax.experimental.pallas import tpu as pltpu

_IOU_T = 0.5
_SCORE_T = 0.05
_MAXDET = 1000
_C = 256

_INTERPRET = False


def _nms_scan_kernel(T, y1r, y2r, x1r, x2r, arr, scr, ctop, opos_ref, kc_ref,
                     keeprows, oacc, kc_s, go_s):
    MAXPAD = oacc.shape[0]

    iota_s = lax.broadcasted_iota(jnp.int32, (_C, _C), 0)
    iota_l = lax.broadcasted_iota(jnp.int32, (_C, _C), 1)
    ident = (iota_s == iota_l).astype(jnp.float32)
    lower_lt = (iota_l < iota_s).astype(jnp.float32)
    lower_le = (iota_l <= iota_s).astype(jnp.float32)

    def t_row(col):
        return jnp.max(ident * col, axis=0, keepdims=True)

    def t_col(row):
        return jnp.max(ident * row, axis=1, keepdims=True)

    def row_slice(ref, t):
        return ref[0, t, :].reshape(1, _C)

    def iou_gt(y1c, y2c, x1c, x2c, ac, t):
        jy1 = row_slice(y1r, t)
        jy2 = row_slice(y2r, t)
        jx1 = row_slice(x1r, t)
        jx2 = row_slice(x2r, t)
        ja = row_slice(arr, t)
        iy1 = jnp.maximum(y1c, jy1)
        iy2 = jnp.minimum(y2c, jy2)
        ix1 = jnp.maximum(x1c, jx1)
        ix2 = jnp.minimum(x2c, jx2)
        inter = jnp.maximum(iy2 - iy1, 0.0) * jnp.maximum(ix2 - ix1, 0.0)
        union = ja + ac - inter
        iou = inter / jnp.maximum(union, 1e-9)
        return (iou > _IOU_T).astype(jnp.float32)

    oacc[...] = jnp.zeros((MAXPAD, 1), jnp.float32)
    kc_s[0] = 0
    go_s[0] = (ctop[0, 0, 0] > _SCORE_T).astype(jnp.int32)

    def chunk_body(t, _):
        @pl.when(go_s[0] == 1)
        def _process():
            kc = kc_s[0]
            y1c = t_col(row_slice(y1r, t))
            y2c = t_col(row_slice(y2r, t))
            x1c = t_col(row_slice(x1r, t))
            x2c = t_col(row_slice(x2r, t))
            ac = t_col(row_slice(arr, t))
            cand_col = t_col((row_slice(scr, t) > _SCORE_T).astype(jnp.float32))

            def cross_body(s, sup):
                snew = jnp.max(iou_gt(y1c, y2c, x1c, x2c, ac, s)
                               * keeprows[s, :].reshape(1, _C),
                               axis=1, keepdims=True)
                return jnp.maximum(sup, snew)

            sup0 = lax.fori_loop(0, t, cross_body,
                                 jnp.zeros((_C, 1), jnp.float32))

            s2 = iou_gt(y1c, y2c, x1c, x2c, ac, t) * lower_lt

            undec0 = cand_col * (1.0 - sup0)

            def fx_cond(c):
                u, _k = c
                return jnp.max(u) > 0.0

            def fx_body(c):
                u, k = c
                act_r = t_row(jnp.maximum(u, k))
                pot = jnp.max(s2 * act_r, axis=1, keepdims=True)
                newly = u * (1.0 - pot)
                k2 = jnp.maximum(k, newly)
                supk = jnp.max(s2 * t_row(k2), axis=1, keepdims=True)
                u2 = u * (1.0 - supk) * (1.0 - newly)
                return u2, k2

            _, keep_col = lax.while_loop(
                fx_cond, fx_body, (undec0, jnp.zeros((_C, 1), jnp.float32)))

            keep_row = t_row(keep_col)
            keeprows[t, :] = keep_row.reshape(_C)

            kcf = kc.astype(jnp.float32)
            rank_col = kcf + jnp.sum(lower_le * keep_row, axis=1,
                                     keepdims=True)
            valid_col = keep_col * (rank_col <= float(_MAXDET)).astype(
                jnp.float32)
            rank_row = t_row(rank_col * valid_col)
            valid_row = t_row(valid_col)
            pos_row = (float(_C) * t.astype(jnp.float32)
                       + lax.broadcasted_iota(jnp.int32, (1, _C), 1)
                       .astype(jnp.float32))
            dest = lax.broadcasted_iota(jnp.int32, (MAXPAD, 1), 0).astype(
                jnp.float32)
            onehot = (dest == (rank_row - 1.0)).astype(jnp.float32) * valid_row
            oacc[...] += jnp.sum(onehot * pos_row, axis=1, keepdims=True)

            n_new = jnp.sum(keep_col).astype(jnp.int32)
            kc_new = kc + n_new
            kc_s[0] = kc_new
            nxt = jnp.minimum(t + 1, T - 1)
            go_s[0] = jnp.where(
                (t + 1 < T) & (kc_new < _MAXDET)
                & (ctop[0, 0, nxt] > _SCORE_T),
                1, 0)

        return _

    lax.fori_loop(0, T, chunk_body, None)

    opos_ref[0, :, :] = oacc[...].astype(jnp.int32)
    kc_ref[0, :, :] = jnp.full((1, 1), kc_s[0], jnp.int32)


def _nms_scan(y1, y2, x1, x2, area, sc, ctop, maxpad):
    B, T, C = y1.shape
    assert C == _C
    grid = (B,)
    row_spec = pl.BlockSpec((1, T, C), lambda i: (i, 0, 0))
    ctop_spec = pl.BlockSpec((1, 1, T), lambda i: (i, 0, 0),
                             memory_space=pltpu.SMEM)
    opos, kcnt = pl.pallas_call(
        functools.partial(_nms_scan_kernel, T),
        grid=grid,
        in_specs=[row_spec] * 6 + [ctop_spec],
        out_specs=[
            pl.BlockSpec((1, maxpad, 1), lambda i: (i, 0, 0)),
            pl.BlockSpec((1, 1, 1), lambda i: (i, 0, 0)),
        ],
        out_shape=[
            jax.ShapeDtypeStruct((B, maxpad, 1), jnp.int32),
            jax.ShapeDtypeStruct((B, 1, 1), jnp.int32),
        ],
        scratch_shapes=[
            pltpu.VMEM((T, C), jnp.float32),
            pltpu.VMEM((maxpad, 1), jnp.float32),
            pltpu.SMEM((1,), jnp.int32),
            pltpu.SMEM((1,), jnp.int32),
        ],
        interpret=_INTERPRET,
    )(y1, y2, x1, x2, area, sc, ctop)
    return opos[:, :, 0], kcnt[:, 0, 0]


def kernel(box_prediction, class_prediction):
    B, N, _ = box_prediction.shape
    scores = jnp.max(jax.nn.softmax(class_prediction, axis=-1), axis=-1)

    sidx = jnp.argsort(-scores, axis=-1)
    s_sc = jnp.take_along_axis(scores, sidx, axis=-1)
    s_box = jnp.take_along_axis(box_prediction, sidx[..., None], axis=1)

    y1 = jnp.minimum(s_box[..., 0], s_box[..., 2])
    y2 = jnp.maximum(s_box[..., 0], s_box[..., 2])
    x1 = jnp.minimum(s_box[..., 1], s_box[..., 3])
    x2 = jnp.maximum(s_box[..., 1], s_box[..., 3])
    area = (y2 - y1) * (x2 - x1)

    T = (N + _C - 1) // _C
    NP = T * _C
    pad = NP - N

    def prep(a, fill):
        return jnp.pad(a, ((0, 0), (0, pad)),
                       constant_values=fill).reshape(B, T, _C)

    y1p, y2p, x1p, x2p, ap = (prep(a, 0.0) for a in (y1, y2, x1, x2, area))
    scp = prep(s_sc, -1.0)
    ctop = scp[:, :, 0].reshape(B, 1, T)

    maxpad = 1024
    opos, kcnt = _nms_scan(y1p, y2p, x1p, x2p, ap, scp, ctop, maxpad)

    opos = opos[:, :_MAXDET]
    valid = (jnp.arange(_MAXDET)[None, :] < kcnt[:, None])
    oidx = jnp.take_along_axis(sidx, opos, axis=-1)
    oidx = jnp.where(valid, oidx, 0)

    m = valid.astype(box_prediction.dtype)[..., None]
    nms_box = jnp.take_along_axis(box_prediction, oidx[..., None], axis=1) * m
    raw_rows = jnp.take_along_axis(class_prediction, oidx[..., None], axis=1)
    nms_cls = jax.nn.softmax(raw_rows, axis=-1) * m
    nms_raw = raw_rows * m
    return nms_box, nms_cls, nms_raw


# speedup vs baseline: 7.6964x; 1.0184x over previous
import functools

import jax
import jax.numpy as jnp
from jax import lax
from jax.experimental import pallas as pl
from jax.experimental.pallas import tpu as pltpu

_IOU_T = 0.5
_SCORE_T = 0.05
_MAXDET = 1000
_C = 256

_INTERPRET = False


def _nms_scan_kernel(T, y1r, y2r, x1r, x2r, arr, scr, ctop, opos_ref, kc_ref,
                     keeprows, oacc, kc_s, go_s):
    MAXPAD = oacc.shape[0]

    iota_s = lax.broadcasted_iota(jnp.int32, (_C, _C), 0)
    iota_l = lax.broadcasted_iota(jnp.int32, (_C, _C), 1)
    ident = (iota_s == iota_l).astype(jnp.float32)
    lower_lt = (iota_l < iota_s).astype(jnp.float32)
    lower_le = (iota_l <= iota_s).astype(jnp.float32)

    def t_row(col):
        return jnp.max(ident * col, axis=0, keepdims=True)

    def t_col(row):
        return jnp.max(ident * row, axis=1, keepdims=True)

    def row_slice(ref, t):
        return ref[0, t, :].reshape(1, _C)

    def iou_gt(y1c, y2c, x1c, x2c, ac, t):
        jy1 = row_slice(y1r, t)
        jy2 = row_slice(y2r, t)
        jx1 = row_slice(x1r, t)
        jx2 = row_slice(x2r, t)
        ja = row_slice(arr, t)
        iy1 = jnp.maximum(y1c, jy1)
        iy2 = jnp.minimum(y2c, jy2)
        ix1 = jnp.maximum(x1c, jx1)
        ix2 = jnp.minimum(x2c, jx2)
        inter = jnp.maximum(iy2 - iy1, 0.0) * jnp.maximum(ix2 - ix1, 0.0)
        union = ja + ac - inter
        iou = inter / jnp.maximum(union, 1e-9)
        return (iou > _IOU_T).astype(jnp.float32)

    oacc[...] = jnp.zeros((MAXPAD, 1), jnp.float32)
    kc_s[0] = 0
    go_s[0] = (ctop[0, 0, 0] > _SCORE_T).astype(jnp.int32)

    def chunk_body(t, _):
        @pl.when(go_s[0] == 1)
        def _process():
            kc = kc_s[0]
            y1c = t_col(row_slice(y1r, t))
            y2c = t_col(row_slice(y2r, t))
            x1c = t_col(row_slice(x1r, t))
            x2c = t_col(row_slice(x2r, t))
            ac = t_col(row_slice(arr, t))
            cand_col = t_col((row_slice(scr, t) > _SCORE_T).astype(jnp.float32))

            def cross_body(s, sup):
                snew = jnp.max(iou_gt(y1c, y2c, x1c, x2c, ac, s)
                               * keeprows[s, :].reshape(1, _C),
                               axis=1, keepdims=True)
                return jnp.maximum(sup, snew)

            sup0 = lax.fori_loop(0, t, cross_body,
                                 jnp.zeros((_C, 1), jnp.float32))

            s2 = iou_gt(y1c, y2c, x1c, x2c, ac, t) * lower_lt

            undec0 = cand_col * (1.0 - sup0)

            def fx_cond(c):
                u, _k = c
                return jnp.max(u) > 0.0

            def fx_body(c):
                u, k = c
                act_r = t_row(jnp.maximum(u, k))
                pot = jnp.max(s2 * act_r, axis=1, keepdims=True)
                newly = u * (1.0 - pot)
                k2 = jnp.maximum(k, newly)
                supk = jnp.max(s2 * t_row(k2), axis=1, keepdims=True)
                u2 = u * (1.0 - supk) * (1.0 - newly)
                return u2, k2

            _, keep_col = lax.while_loop(
                fx_cond, fx_body, (undec0, jnp.zeros((_C, 1), jnp.float32)))

            keep_row = t_row(keep_col)
            keeprows[t, :] = keep_row.reshape(_C)

            kcf = kc.astype(jnp.float32)
            rank_col = kcf + jnp.sum(lower_le * keep_row, axis=1,
                                     keepdims=True)
            valid_col = keep_col * (rank_col <= float(_MAXDET)).astype(
                jnp.float32)
            rank_row = t_row(rank_col * valid_col)
            valid_row = t_row(valid_col)
            pos_row = (float(_C) * t.astype(jnp.float32)
                       + lax.broadcasted_iota(jnp.int32, (1, _C), 1)
                       .astype(jnp.float32))
            dest = lax.broadcasted_iota(jnp.int32, (MAXPAD, 1), 0).astype(
                jnp.float32)
            onehot = (dest == (rank_row - 1.0)).astype(jnp.float32) * valid_row
            oacc[...] += jnp.sum(onehot * pos_row, axis=1, keepdims=True)

            n_new = jnp.sum(keep_col).astype(jnp.int32)
            kc_new = kc + n_new
            kc_s[0] = kc_new
            nxt = jnp.minimum(t + 1, T - 1)
            go_s[0] = jnp.where(
                (t + 1 < T) & (kc_new < _MAXDET)
                & (ctop[0, 0, nxt] > _SCORE_T),
                1, 0)

        return _

    lax.fori_loop(0, T, chunk_body, None)

    opos_ref[0, :, :] = oacc[...].astype(jnp.int32)
    kc_ref[0, :, :] = jnp.full((1, 1), kc_s[0], jnp.int32)


def _nms_scan(y1, y2, x1, x2, area, sc, ctop, maxpad):
    B, T, C = y1.shape
    assert C == _C
    grid = (B,)
    row_spec = pl.BlockSpec((1, T, C), lambda i: (i, 0, 0))
    ctop_spec = pl.BlockSpec((1, 1, T), lambda i: (i, 0, 0),
                             memory_space=pltpu.SMEM)
    opos, kcnt = pl.pallas_call(
        functools.partial(_nms_scan_kernel, T),
        grid=grid,
        in_specs=[row_spec] * 6 + [ctop_spec],
        out_specs=[
            pl.BlockSpec((1, maxpad, 1), lambda i: (i, 0, 0)),
            pl.BlockSpec((1, 1, 1), lambda i: (i, 0, 0)),
        ],
        out_shape=[
            jax.ShapeDtypeStruct((B, maxpad, 1), jnp.int32),
            jax.ShapeDtypeStruct((B, 1, 1), jnp.int32),
        ],
        scratch_shapes=[
            pltpu.VMEM((T, C), jnp.float32),
            pltpu.VMEM((maxpad, 1), jnp.float32),
            pltpu.SMEM((1,), jnp.int32),
            pltpu.SMEM((1,), jnp.int32),
        ],
        interpret=_INTERPRET,
    )(y1, y2, x1, x2, area, sc, ctop)
    return opos[:, :, 0], kcnt[:, 0, 0]


def kernel(box_prediction, class_prediction):
    B, N, _ = box_prediction.shape
    scores = jnp.max(jax.nn.softmax(class_prediction, axis=-1), axis=-1)

    uy1 = jnp.minimum(box_prediction[..., 0], box_prediction[..., 2])
    uy2 = jnp.maximum(box_prediction[..., 0], box_prediction[..., 2])
    ux1 = jnp.minimum(box_prediction[..., 1], box_prediction[..., 3])
    ux2 = jnp.maximum(box_prediction[..., 1], box_prediction[..., 3])
    uarea = (uy2 - uy1) * (ux2 - ux1)
    iota = jnp.broadcast_to(jnp.arange(N, dtype=jnp.int32), (B, N))

    _, s_sc, y1, y2, x1, x2, area, sidx = lax.sort(
        (-scores, scores, uy1, uy2, ux1, ux2, uarea, iota),
        num_keys=1, is_stable=True)

    T = (N + _C - 1) // _C
    NP = T * _C
    pad = NP - N

    def prep(a, fill):
        return jnp.pad(a, ((0, 0), (0, pad)),
                       constant_values=fill).reshape(B, T, _C)

    y1p, y2p, x1p, x2p, ap = (prep(a, 0.0) for a in (y1, y2, x1, x2, area))
    scp = prep(s_sc, -1.0)
    ctop = scp[:, :, 0].reshape(B, 1, T)

    maxpad = 1024
    opos, kcnt = _nms_scan(y1p, y2p, x1p, x2p, ap, scp, ctop, maxpad)

    opos = opos[:, :_MAXDET]
    valid = (jnp.arange(_MAXDET)[None, :] < kcnt[:, None])
    oidx = jnp.take_along_axis(sidx, opos, axis=-1)
    oidx = jnp.where(valid, oidx, 0)

    m = valid.astype(box_prediction.dtype)[..., None]
    nms_box = jnp.take_along_axis(box_prediction, oidx[..., None], axis=1) * m
    raw_rows = jnp.take_along_axis(class_prediction, oidx[..., None], axis=1)
    nms_cls = jax.nn.softmax(raw_rows, axis=-1) * m
    nms_raw = raw_rows * m
    return nms_box, nms_cls, nms_raw
